# baseline (speedup 1.0000x reference)
import functools

import jax
import jax.numpy as jnp
from jax import lax
from jax.experimental import pallas as pl
from jax.experimental.pallas import tpu as pltpu
from jax.experimental.pallas import tpu_sc as plsc

MM, UU, VV, NN = 8, 400, 400, 1048576
NC, NS, L = 2, 16, 16
NW = NC * NS
PW = NN // NW
C = 512
SB = 128
NGS = C // SB
GPS = SB // L
NCHUNK = PW // C


def _sc_body(x_hbm, m_hbm, u_hbm, v_hbm, mp_hbm, bp_hbm, out_hbm,
             u_v, v_v, m_v, x_v, ir_v, jr_v,
             i11_v, i21_v, i12_v, i22_v,
             m11_v, m21_v, m12_v, m22_v,
             b11_v, b21_v, b12_v, b22_v,
             out_v, sem):
    wid = lax.axis_index("s") * NC + lax.axis_index("c")
    iota = lax.iota(jnp.int32, L)

    def chunk_body(c, _):
        base = wid * PW + c * C
        pltpu.sync_copy(u_hbm.at[pl.ds(base, C)], u_v)
        pltpu.sync_copy(v_hbm.at[pl.ds(base, C)], v_v)
        pltpu.sync_copy(m_hbm.at[pl.ds(base, C)], m_v)
        pltpu.sync_copy(x_hbm.at[pl.ds(base, C), :], x_v)

        def idx_body(g, _):
            sl = pl.ds(g * L, L)
            uu = u_v[sl] * jnp.float32(UU)
            vv = v_v[sl] * jnp.float32(VV)
            uu = jnp.where(uu == jnp.float32(UU), jnp.float32(UU - 1), uu)
            vv = jnp.where(vv == jnp.float32(VV), jnp.float32(VV - 1), vv)
            iu1 = uu.astype(jnp.int32)
            iv1 = vv.astype(jnp.int32)
            ir_v[sl] = uu - iu1.astype(jnp.float32)
            jr_v[sl] = vv - iv1.astype(jnp.float32)
            iu2 = jnp.where(iu1 == UU - 1, 0, iu1 + 1)
            iv2 = jnp.where(iv1 == VV - 1, 0, iv1 + 1)
            mb = m_v[sl] * jnp.int32(UU * VV)
            r1 = mb + iu1 * VV
            r2 = mb + iu2 * VV
            k = g // GPS
            ssl = pl.ds((g % GPS) * L, L)
            i11_v[k, ssl] = r1 + iv1
            i21_v[k, ssl] = r2 + iv1
            i12_v[k, ssl] = r1 + iv2
            i22_v[k, ssl] = r2 + iv2
            return _

        lax.fori_loop(0, C // L, idx_body, None)

        descs = []
        for k in range(NGS):
            descs.append(pltpu.async_copy(mp_hbm.at[i11_v.at[k]], m11_v.at[k], sem))
            descs.append(pltpu.async_copy(mp_hbm.at[i21_v.at[k]], m21_v.at[k], sem))
            descs.append(pltpu.async_copy(mp_hbm.at[i12_v.at[k]], m12_v.at[k], sem))
            descs.append(pltpu.async_copy(mp_hbm.at[i22_v.at[k]], m22_v.at[k], sem))
            descs.append(pltpu.async_copy(bp_hbm.at[i11_v.at[k]], b11_v.at[k], sem))
            descs.append(pltpu.async_copy(bp_hbm.at[i21_v.at[k]], b21_v.at[k], sem))
            descs.append(pltpu.async_copy(bp_hbm.at[i12_v.at[k]], b12_v.at[k], sem))
            descs.append(pltpu.async_copy(bp_hbm.at[i22_v.at[k]], b22_v.at[k], sem))
        for d in descs:
            d.wait()

        def blend_body(k, _):
            def g_body(go, _):
                sl = pl.ds(k * SB + go * L, L)
                ir = ir_v[sl]
                jr = jr_v[sl]
                one = jnp.float32(1)
                w11 = (one - ir) * (one - jr)
                w21 = ir * (one - jr)
                w12 = (one - ir) * jr
                w22 = ir * jr
                p = go * L + iota
                pg = k * SB + go * L + iota
                kv = iota * 0 + k
                c0 = jnp.full((L,), 0, jnp.int32)
                c1 = jnp.full((L,), 1, jnp.int32)
                c2 = jnp.full((L,), 2, jnp.int32)
                x0 = plsc.load_gather(x_v, [pg, c0])
                x1 = plsc.load_gather(x_v, [pg, c1])
                x2 = plsc.load_gather(x_v, [pg, c2])
                vals = []
                for e in range(9):
                    ec = jnp.full((L,), e, jnp.int32)
                    g11 = plsc.load_gather(m11_v, [kv, p, ec])
                    g21 = plsc.load_gather(m21_v, [kv, p, ec])
                    g12 = plsc.load_gather(m12_v, [kv, p, ec])
                    g22 = plsc.load_gather(m22_v, [kv, p, ec])
                    vals.append(w11 * g11 + w21 * g21 + w12 * g12 + w22 * g22)
                for e in range(3):
                    ec = jnp.full((L,), e, jnp.int32)
                    g11 = plsc.load_gather(b11_v, [kv, p, ec])
                    g21 = plsc.load_gather(b21_v, [kv, p, ec])
                    g12 = plsc.load_gather(b12_v, [kv, p, ec])
                    g22 = plsc.load_gather(b22_v, [kv, p, ec])
                    vals.append(w11 * g11 + w21 * g21 + w12 * g12 + w22 * g22)
                o0 = x0 * vals[0] + x1 * vals[3] + x2 * vals[6] + vals[9]
                o1 = x0 * vals[1] + x1 * vals[4] + x2 * vals[7] + vals[10]
                o2 = x0 * vals[2] + x1 * vals[5] + x2 * vals[8] + vals[11]
                plsc.store_scatter(out_v, [pg, c0], o0)
                plsc.store_scatter(out_v, [pg, c1], o1)
                plsc.store_scatter(out_v, [pg, c2], o2)
                return _

            lax.fori_loop(0, GPS, g_body, None)
            return _

        lax.fori_loop(0, NGS, blend_body, None)
        pltpu.sync_copy(out_v, out_hbm.at[pl.ds(base, C), :])
        return _

    lax.fori_loop(0, NCHUNK, chunk_body, None)


@jax.jit
def _run(x, m, u, v, mp, bp):
    mesh = plsc.VectorSubcoreMesh(core_axis_name="c", subcore_axis_name="s")
    f = functools.partial(
        pl.kernel,
        out_type=jax.ShapeDtypeStruct((NN, 3), jnp.float32),
        mesh=mesh,
        compiler_params=pltpu.CompilerParams(
            needs_layout_passes=False, use_tc_tiling_on_sc=False),
        scratch_types=[
            pltpu.VMEM((C,), jnp.float32),
            pltpu.VMEM((C,), jnp.float32),
            pltpu.VMEM((C,), jnp.int32),
            pltpu.VMEM((C, 3), jnp.float32),
            pltpu.VMEM((C,), jnp.float32),
            pltpu.VMEM((C,), jnp.float32),
            pltpu.VMEM((NGS, SB), jnp.int32),
            pltpu.VMEM((NGS, SB), jnp.int32),
            pltpu.VMEM((NGS, SB), jnp.int32),
            pltpu.VMEM((NGS, SB), jnp.int32),
            pltpu.VMEM((NGS, SB, 9), jnp.float32),
            pltpu.VMEM((NGS, SB, 9), jnp.float32),
            pltpu.VMEM((NGS, SB, 9), jnp.float32),
            pltpu.VMEM((NGS, SB, 9), jnp.float32),
            pltpu.VMEM((NGS, SB, 3), jnp.float32),
            pltpu.VMEM((NGS, SB, 3), jnp.float32),
            pltpu.VMEM((NGS, SB, 3), jnp.float32),
            pltpu.VMEM((NGS, SB, 3), jnp.float32),
            pltpu.VMEM((C, 3), jnp.float32),
            pltpu.SemaphoreType.DMA,
        ],
    )(_sc_body)
    return f(x, m, u, v, mp, bp)


def kernel(x, m, u, v, m_param, b_param):
    muv = MM * UU * VV
    return _run(x, m, u, v,
                m_param.reshape(muv, 9), b_param.reshape(muv, 3))


# speedup vs baseline: 2.6697x; 2.6697x over previous
import functools

import jax
import jax.numpy as jnp
from jax import lax
from jax.experimental import pallas as pl
from jax.experimental.pallas import tpu as pltpu
from jax.experimental.pallas import tpu_sc as plsc

MM, UU, VV, NN = 8, 400, 400, 1048576
MUV = MM * UU * VV
NC, NS, L = 2, 16, 16
NW = NC * NS
PW = NN // NW
C = 512
SB = 128
NGS = C // SB
GPS = SB // L
NCHUNK = PW // C
KB = UU * VV // 128
KPW = (KB + NW - 1) // NW

_params = pltpu.CompilerParams(
    needs_layout_passes=False, use_tc_tiling_on_sc=False)


def _pack_body(mp_hbm, bp_hbm, tbl_hbm, m_st, b_st, pack_v, sem_i, sem_o):
    wid = lax.axis_index("s") * NC + lax.axis_index("c")
    iota = lax.iota(jnp.int32, L)

    def kb_body(i, _):
        kb = wid * KPW + i

        @pl.when(kb < KB)
        def _do():
            t0 = kb * 128
            descs = []
            for mi in range(3):
                for mj in range(3):
                    descs.append(pltpu.async_copy(
                        mp_hbm.at[mi, mj, kb], m_st.at[mi * 3 + mj], sem_i))
            for mat in range(MM):
                for bj in range(3):
                    descs.append(pltpu.async_copy(
                        bp_hbm.at[mat, bj, pl.ds(t0, 128)], b_st.at[mat, bj],
                        sem_i))
            for d in descs:
                d.wait()

            def mat_body(mat, _):
                mv = iota * 0 + mat
                for e in range(9):
                    ec = jnp.full((L,), e, jnp.int32)
                    for go in range(8):
                        vec = m_st[e, mat, pl.ds(go * L, L)]
                        plsc.store_scatter(pack_v, [mv, go * L + iota, ec], vec)
                for bj in range(3):
                    ec = jnp.full((L,), 9 + bj, jnp.int32)
                    for go in range(8):
                        vec = b_st[mat, bj, pl.ds(go * L, L)]
                        plsc.store_scatter(pack_v, [mv, go * L + iota, ec], vec)
                return _

            lax.fori_loop(0, MM, mat_body, None)

            odescs = []
            for mat in range(MM):
                odescs.append(pltpu.async_copy(
                    pack_v.at[mat],
                    tbl_hbm.at[pl.ds(mat * (UU * VV) + t0, 128), :], sem_o))
            for d in odescs:
                d.wait()

        return _

    lax.fori_loop(0, KPW, kb_body, None)


def _gather_body(x_hbm, m_hbm, u_hbm, v_hbm, tbl_hbm, out_hbm,
                 u_v, v_v, m_v, x_v, ir_v, jr_v,
                 i11_v, i21_v, i12_v, i22_v,
                 r11_v, r21_v, r12_v, r22_v,
                 out_v, sem):
    wid = lax.axis_index("s") * NC + lax.axis_index("c")
    iota = lax.iota(jnp.int32, L)

    def chunk_body(c, _):
        base = wid * PW + c * C
        pltpu.sync_copy(u_hbm.at[pl.ds(base, C)], u_v)
        pltpu.sync_copy(v_hbm.at[pl.ds(base, C)], v_v)
        pltpu.sync_copy(m_hbm.at[pl.ds(base, C)], m_v)
        pltpu.sync_copy(x_hbm.at[pl.ds(base, C), :], x_v)

        def idx_body(g, _):
            sl = pl.ds(g * L, L)
            uu = u_v[sl] * jnp.float32(UU)
            vv = v_v[sl] * jnp.float32(VV)
            uu = jnp.where(uu == jnp.float32(UU), jnp.float32(UU - 1), uu)
            vv = jnp.where(vv == jnp.float32(VV), jnp.float32(VV - 1), vv)
            iu1 = uu.astype(jnp.int32)
            iv1 = vv.astype(jnp.int32)
            ir_v[sl] = uu - iu1.astype(jnp.float32)
            jr_v[sl] = vv - iv1.astype(jnp.float32)
            iu2 = jnp.where(iu1 == UU - 1, 0, iu1 + 1)
            iv2 = jnp.where(iv1 == VV - 1, 0, iv1 + 1)
            mb = m_v[sl] * jnp.int32(UU * VV)
            r1 = mb + iu1 * VV
            r2 = mb + iu2 * VV
            k = g // GPS
            ssl = pl.ds((g % GPS) * L, L)
            i11_v[k, ssl] = r1 + iv1
            i21_v[k, ssl] = r2 + iv1
            i12_v[k, ssl] = r1 + iv2
            i22_v[k, ssl] = r2 + iv2
            return _

        lax.fori_loop(0, C // L, idx_body, None)

        descs = []
        for k in range(NGS):
            descs.append(pltpu.async_copy(tbl_hbm.at[i11_v.at[k]], r11_v.at[k], sem))
            descs.append(pltpu.async_copy(tbl_hbm.at[i21_v.at[k]], r21_v.at[k], sem))
            descs.append(pltpu.async_copy(tbl_hbm.at[i12_v.at[k]], r12_v.at[k], sem))
            descs.append(pltpu.async_copy(tbl_hbm.at[i22_v.at[k]], r22_v.at[k], sem))
        for d in descs:
            d.wait()

        def blend_body(k, _):
            def g_body(go, _):
                sl = pl.ds(k * SB + go * L, L)
                ir = ir_v[sl]
                jr = jr_v[sl]
                one = jnp.float32(1)
                w11 = (one - ir) * (one - jr)
                w21 = ir * (one - jr)
                w12 = (one - ir) * jr
                w22 = ir * jr
                p = go * L + iota
                pg = k * SB + go * L + iota
                kv = iota * 0 + k
                c0 = jnp.full((L,), 0, jnp.int32)
                c1 = jnp.full((L,), 1, jnp.int32)
                c2 = jnp.full((L,), 2, jnp.int32)
                x0 = plsc.load_gather(x_v, [pg, c0])
                x1 = plsc.load_gather(x_v, [pg, c1])
                x2 = plsc.load_gather(x_v, [pg, c2])
                vals = []
                for e in range(12):
                    ec = jnp.full((L,), e, jnp.int32)
                    g11 = plsc.load_gather(r11_v, [kv, p, ec])
                    g21 = plsc.load_gather(r21_v, [kv, p, ec])
                    g12 = plsc.load_gather(r12_v, [kv, p, ec])
                    g22 = plsc.load_gather(r22_v, [kv, p, ec])
                    vals.append(w11 * g11 + w21 * g21 + w12 * g12 + w22 * g22)
                o0 = x0 * vals[0] + x1 * vals[3] + x2 * vals[6] + vals[9]
                o1 = x0 * vals[1] + x1 * vals[4] + x2 * vals[7] + vals[10]
                o2 = x0 * vals[2] + x1 * vals[5] + x2 * vals[8] + vals[11]
                plsc.store_scatter(out_v, [pg, c0], o0)
                plsc.store_scatter(out_v, [pg, c1], o1)
                plsc.store_scatter(out_v, [pg, c2], o2)
                return _

            lax.fori_loop(0, GPS, g_body, None)
            return _

        lax.fori_loop(0, NGS, blend_body, None)
        pltpu.sync_copy(out_v, out_hbm.at[pl.ds(base, C), :])
        return _

    lax.fori_loop(0, NCHUNK, chunk_body, None)


@jax.jit
def _run(x, m, u, v, mp_view, bp_view):
    mesh = plsc.VectorSubcoreMesh(core_axis_name="c", subcore_axis_name="s")
    pack = functools.partial(
        pl.kernel,
        out_type=jax.ShapeDtypeStruct((MUV, 16), jnp.float32),
        mesh=mesh,
        compiler_params=_params,
        scratch_types=[
            pltpu.VMEM((9, MM, 128), jnp.float32),
            pltpu.VMEM((MM, 3, 128), jnp.float32),
            pltpu.VMEM((MM, 128, 16), jnp.float32),
            pltpu.SemaphoreType.DMA,
            pltpu.SemaphoreType.DMA,
        ],
    )(_pack_body)
    tbl = pack(mp_view, bp_view)

    gather = functools.partial(
        pl.kernel,
        out_type=jax.ShapeDtypeStruct((NN, 3), jnp.float32),
        mesh=mesh,
        compiler_params=_params,
        scratch_types=[
            pltpu.VMEM((C,), jnp.float32),
            pltpu.VMEM((C,), jnp.float32),
            pltpu.VMEM((C,), jnp.int32),
            pltpu.VMEM((C, 3), jnp.float32),
            pltpu.VMEM((C,), jnp.float32),
            pltpu.VMEM((C,), jnp.float32),
            pltpu.VMEM((NGS, SB), jnp.int32),
            pltpu.VMEM((NGS, SB), jnp.int32),
            pltpu.VMEM((NGS, SB), jnp.int32),
            pltpu.VMEM((NGS, SB), jnp.int32),
            pltpu.VMEM((NGS, SB, 16), jnp.float32),
            pltpu.VMEM((NGS, SB, 16), jnp.float32),
            pltpu.VMEM((NGS, SB, 16), jnp.float32),
            pltpu.VMEM((NGS, SB, 16), jnp.float32),
            pltpu.VMEM((C, 3), jnp.float32),
            pltpu.SemaphoreType.DMA,
        ],
    )(_gather_body)
    return gather(x, m, u, v, tbl)


def kernel(x, m, u, v, m_param, b_param):
    mp_view = (m_param.transpose(2, 3, 0, 1)
               .reshape(3, 3, MM, KB, 128)
               .transpose(0, 1, 3, 2, 4))
    bp_view = b_param.transpose(0, 3, 2, 1).reshape(MM, 3, UU * VV)
    return _run(x, m, u, v, mp_view, bp_view)
